# baseline (speedup 1.0000x reference)
import functools

import jax
import jax.numpy as jnp
from jax import lax
from jax.experimental import pallas as pl
from jax.experimental.pallas import tpu as pltpu
from jax.experimental.pallas import tpu_sc as plsc

N_NODES = 10000
N_EDGES = 320000
N_Z = 128
EDGE_DIM = 16

_info = plsc.get_sparse_core_info()
NC, NS = _info.num_cores, _info.num_subcores
NW = NC * NS
EDGES_PER_W = N_EDGES // NW
CHUNK = 1000
N_CHUNKS = EDGES_PER_W // CHUNK


def _tables_body(z_ref, w1_ref, w2_ref, b_ref, t1_ref, t2_ref):
    z = z_ref[...]
    dn = (((1,), (1,)), ((), ()))
    t1_ref[...] = (
        jax.lax.dot_general(z, w1_ref[...], dn, preferred_element_type=jnp.float32)
        + b_ref[...]
    )
    t2_ref[...] = jax.lax.dot_general(
        z, w2_ref[...], dn, preferred_element_type=jnp.float32
    )


def _make_tables(z, W1, W2, b2d):
    return pl.pallas_call(
        _tables_body,
        out_shape=[
            jax.ShapeDtypeStruct((N_NODES, EDGE_DIM), jnp.float32),
            jax.ShapeDtypeStruct((N_NODES, EDGE_DIM), jnp.float32),
        ],
    )(z, W1, W2, b2d)


@functools.partial(
    pl.kernel,
    out_type=jax.ShapeDtypeStruct((N_EDGES * EDGE_DIM // 128, 128), jnp.float32),
    mesh=plsc.VectorSubcoreMesh(core_axis_name="c", subcore_axis_name="s"),
    compiler_params=pltpu.CompilerParams(use_tc_tiling_on_sc=False),
    scratch_types=[
        pltpu.VMEM((2, CHUNK), jnp.int32),
        pltpu.VMEM((2, CHUNK), jnp.int32),
        pltpu.VMEM((CHUNK, EDGE_DIM), jnp.float32),
        pltpu.VMEM((CHUNK, EDGE_DIM), jnp.float32),
        pltpu.VMEM((CHUNK, EDGE_DIM), jnp.float32),
        pltpu.VMEM((CHUNK, EDGE_DIM), jnp.float32),
        pltpu.VMEM((2, CHUNK * EDGE_DIM // 128, 128), jnp.float32),
        pltpu.SemaphoreType.DMA,
        pltpu.SemaphoreType.DMA,
        pltpu.SemaphoreType.DMA,
        pltpu.SemaphoreType.DMA,
    ],
)
def _edge_gather_add(t1_hbm, t2_hbm, src_hbm, dst_hbm, out_hbm,
                     idx1, idx2, r1a, r2a, r1b, r2b, obuf,
                     sem1a, sem2a, sem1b, sem2b):
    wid = lax.axis_index("s") * NC + lax.axis_index("c")
    base = wid * EDGES_PER_W
    r1 = (r1a, r1b)
    r2 = (r2a, r2b)
    sems = ((sem1a, sem2a), (sem1b, sem2b))

    def issue(c, buf):
        off = base + c * CHUNK
        pltpu.sync_copy(src_hbm.at[pl.ds(off, CHUNK)], idx1.at[buf])
        pltpu.sync_copy(dst_hbm.at[pl.ds(off, CHUNK)], idx2.at[buf])
        cp1 = pltpu.async_copy(t1_hbm.at[idx1.at[buf]], r1[buf], sems[buf][0])
        cp2 = pltpu.async_copy(t2_hbm.at[idx2.at[buf]], r2[buf], sems[buf][1])
        return cp1, cp2

    pending = issue(0, 0)
    for c in range(N_CHUNKS):
        buf = c % 2
        if c + 1 < N_CHUNKS:
            nxt = issue(c + 1, (c + 1) % 2)
        pending[0].wait()
        pending[1].wait()

        def row_body(i8, carry, a=r1[buf], b=r2[buf], o=obuf.at[buf]):
            for k in range(8):
                o[i8, pl.ds(k * EDGE_DIM, EDGE_DIM)] = (
                    a[i8 * 8 + k, :] + b[i8 * 8 + k, :]
                )
            return carry

        lax.fori_loop(0, CHUNK // 8, row_body, 0, unroll=2)
        off128 = (base + c * CHUNK) * EDGE_DIM // 128
        pltpu.sync_copy(obuf.at[buf],
                        out_hbm.at[pl.ds(off128, CHUNK * EDGE_DIM // 128)])
        if c + 1 < N_CHUNKS:
            pending = nxt


def kernel(z, edge_index, W, b):
    edge_index = edge_index.astype(jnp.int32)
    W1 = W[:, :N_Z]
    W2 = W[:, N_Z:]
    t1, t2 = _make_tables(z, W1, W2, b.reshape(1, EDGE_DIM))
    flat = _edge_gather_add(t1, t2, edge_index[0], edge_index[1])
    return flat.reshape(N_EDGES, EDGE_DIM)
